# baseline (device time: 64387 ns/iter reference)
import jax
import jax.numpy as jnp
from jax import lax
from jax.experimental import pallas as pl
from jax.experimental.pallas import tpu as pltpu

N_DEV = 32
ROW = 16
R_HOPS = ROW // 2
L_HOPS = ROW // 2 - 1


def kernel(q, k, v):
    s_per, d = q.shape
    scale = 1.0 / (d ** 0.5)

    def body(q_ref, k_ref, v_ref, out_ref,
             qmine, qother, rbuf, lbuf, sacc, racc,
             q_sems, br_send, br_recv, bl_send, bl_recv, c_sems):
        p = lax.axis_index("i")
        base = (p // ROW) * ROW
        w = p % ROW
        right = base + (w + 1) % ROW
        left = base + (w - 1) % ROW
        partner = (p + ROW) % N_DEV

        barrier_sem = pltpu.get_barrier_semaphore()
        for nbr in [left, right, partner]:
            pl.semaphore_signal(
                barrier_sem, inc=1,
                device_id=(nbr,), device_id_type=pl.DeviceIdType.MESH,
            )
        pl.semaphore_wait(barrier_sem, 3)

        k_bf = k_ref[:, :].astype(jnp.bfloat16)
        v_bf = v_ref[:, :].astype(jnp.bfloat16)
        rbuf[0, :s_per, :] = k_bf
        rbuf[0, s_per:, :] = v_bf
        lbuf[0, :s_per, :] = k_bf
        lbuf[0, s_per:, :] = v_bf
        qmine[:, :] = (q_ref[:, :] * scale).astype(jnp.bfloat16)

        def make(src, dst, send, recv, dev):
            return pltpu.make_async_remote_copy(
                src_ref=src, dst_ref=dst, send_sem=send, recv_sem=recv,
                device_id=(dev,), device_id_type=pl.DeviceIdType.MESH,
            )

        q_desc = make(qmine, qother, q_sems.at[0], q_sems.at[1], partner)
        r_desc = [
            make(rbuf.at[h], rbuf.at[h + 1], br_send.at[h + 1],
                 br_recv.at[h + 1], right)
            for h in range(R_HOPS)
        ]
        l_desc = [
            make(lbuf.at[h], lbuf.at[h + 1], bl_send.at[h + 1],
                 bl_recv.at[h + 1], left)
            for h in range(L_HOPS)
        ]
        c_desc = make(sacc, racc, c_sems.at[0], c_sems.at[1], partner)

        q_desc.start()
        r_desc[0].start()
        l_desc[0].start()

        q_desc.wait_recv()

        q_all = jnp.concatenate([qmine[:, :], qother[:, :]], axis=0)
        ones = jnp.ones((s_per, d), dtype=jnp.bfloat16)
        acc = jnp.zeros((2 * s_per, 2 * d), dtype=jnp.float32)

        def fold(acc, buf, slot):
            k_h = buf[slot, :s_per, :]
            v_aug = jnp.concatenate([buf[slot, s_per:, :], ones], axis=1)
            scores = jax.lax.dot_general(
                q_all, k_h,
                (((1,), (1,)), ((), ())),
                preferred_element_type=jnp.float32,
            )
            pr = jnp.exp(scores).astype(jnp.bfloat16)
            pv = jax.lax.dot_general(
                pr, v_aug,
                (((1,), (0,)), ((), ())),
                preferred_element_type=jnp.float32,
            )
            return acc + pv

        acc = fold(acc, rbuf, 0)

        for h in range(1, R_HOPS + 1):
            r_desc[h - 1].wait_recv()
            if h < R_HOPS:
                r_desc[h].start()
            acc = fold(acc, rbuf, h)
            if h <= L_HOPS:
                l_desc[h - 1].wait_recv()
                if h < L_HOPS:
                    l_desc[h].start()
                acc = fold(acc, lbuf, h)

        sacc[:, :] = acc[s_per:, :].astype(jnp.bfloat16)
        c_desc.start()
        c_desc.wait_recv()

        aug0 = acc[:s_per, :] + racc[:, :].astype(jnp.float32)
        out_ref[:, :] = aug0[:, :d] / aug0[:, d:d + 1]

        for desc in [q_desc, c_desc] + r_desc + l_desc:
            desc.wait_send()

    return pl.pallas_call(
        body,
        out_shape=jax.ShapeDtypeStruct((s_per, d), jnp.float32),
        in_specs=[
            pl.BlockSpec(memory_space=pltpu.VMEM),
            pl.BlockSpec(memory_space=pltpu.VMEM),
            pl.BlockSpec(memory_space=pltpu.VMEM),
        ],
        out_specs=pl.BlockSpec(memory_space=pltpu.VMEM),
        scratch_shapes=[
            pltpu.VMEM((s_per, d), jnp.bfloat16),
            pltpu.VMEM((s_per, d), jnp.bfloat16),
            pltpu.VMEM((R_HOPS + 1, 2 * s_per, d), jnp.bfloat16),
            pltpu.VMEM((L_HOPS + 1, 2 * s_per, d), jnp.bfloat16),
            pltpu.VMEM((s_per, 2 * d), jnp.bfloat16),
            pltpu.VMEM((s_per, 2 * d), jnp.bfloat16),
            pltpu.SemaphoreType.DMA((2,)),
            pltpu.SemaphoreType.DMA((R_HOPS + 1,)),
            pltpu.SemaphoreType.DMA((R_HOPS + 1,)),
            pltpu.SemaphoreType.DMA((L_HOPS + 1,)),
            pltpu.SemaphoreType.DMA((L_HOPS + 1,)),
            pltpu.SemaphoreType.DMA((2,)),
        ],
        compiler_params=pltpu.CompilerParams(collective_id=0),
    )(q, k, v)


# device time: 62435 ns/iter; 1.0313x vs baseline; 1.0313x over previous
import jax
import jax.numpy as jnp
from jax import lax
from jax.experimental import pallas as pl
from jax.experimental.pallas import tpu as pltpu

N_DEV = 32
ROW = 16
R_HOPS = ROW // 2
L_HOPS = ROW // 2 - 1


def kernel(q, k, v):
    s_per, d = q.shape
    scale = 1.0 / (d ** 0.5)

    def body(q_ref, k_ref, v_ref, out_ref,
             qmine, qother, rbuf, lbuf, sacc, racc,
             q_sems, br_send, br_recv, bl_send, bl_recv, c_sems):
        p = lax.axis_index("i")
        base = (p // ROW) * ROW
        w = p % ROW
        right = base + (w + 1) % ROW
        left = base + (w - 1) % ROW
        partner = (p + ROW) % N_DEV

        barrier_sem = pltpu.get_barrier_semaphore()
        for nbr in [left, right, partner]:
            pl.semaphore_signal(
                barrier_sem, inc=1,
                device_id=(nbr,), device_id_type=pl.DeviceIdType.MESH,
            )
        pl.semaphore_wait(barrier_sem, 3)

        k_bf = k_ref[:, :].astype(jnp.bfloat16)
        v_bf = v_ref[:, :].astype(jnp.bfloat16)
        rbuf[0, :s_per, :] = k_bf
        rbuf[0, s_per:, :] = v_bf
        lbuf[0, :s_per, :] = k_bf
        lbuf[0, s_per:, :] = v_bf
        qmine[:, :] = (q_ref[:, :] * scale).astype(jnp.bfloat16)

        def make(src, dst, send, recv, dev):
            return pltpu.make_async_remote_copy(
                src_ref=src, dst_ref=dst, send_sem=send, recv_sem=recv,
                device_id=(dev,), device_id_type=pl.DeviceIdType.MESH,
            )

        q_desc = make(qmine, qother, q_sems.at[0], q_sems.at[1], partner)
        r_desc = [
            make(rbuf.at[h], rbuf.at[h + 1], br_send.at[h + 1],
                 br_recv.at[h + 1], right)
            for h in range(R_HOPS)
        ]
        l_desc = [
            make(lbuf.at[h], lbuf.at[h + 1], bl_send.at[h + 1],
                 bl_recv.at[h + 1], left)
            for h in range(L_HOPS)
        ]
        c_desc = make(sacc, racc, c_sems.at[0], c_sems.at[1], partner)

        q_desc.start()
        r_desc[0].start()
        l_desc[0].start()

        q_desc.wait_recv()

        q_all = jnp.concatenate([qmine[:, :], qother[:, :]], axis=0)
        ones = jnp.ones((s_per, d), dtype=jnp.bfloat16)
        acc = jnp.zeros((2 * s_per, 2 * d), dtype=jnp.float32)

        def fold(acc, buf, slot):
            k_h = buf[slot, :s_per, :]
            v_aug = jnp.concatenate([buf[slot, s_per:, :], ones], axis=1)
            scores = jax.lax.dot_general(
                q_all, k_h,
                (((1,), (1,)), ((), ())),
                preferred_element_type=jnp.float32,
            )
            pr = jnp.exp(scores).astype(jnp.bfloat16)
            pv = jax.lax.dot_general(
                pr, v_aug,
                (((1,), (0,)), ((), ())),
                preferred_element_type=jnp.float32,
            )
            return acc + pv

        acc = fold(acc, rbuf, 0)
        acc = fold(acc, rbuf, 0)

        for h in range(1, R_HOPS + 1):
            r_desc[h - 1].wait_recv()
            if h < R_HOPS:
                r_desc[h].start()
            acc = fold(acc, rbuf, h)
            acc = fold(acc, rbuf, h)
            if h <= L_HOPS:
                l_desc[h - 1].wait_recv()
                if h < L_HOPS:
                    l_desc[h].start()
                acc = fold(acc, lbuf, h)
                acc = fold(acc, lbuf, h)

        sacc[:, :] = acc[s_per:, :].astype(jnp.bfloat16)
        c_desc.start()
        c_desc.wait_recv()

        aug0 = acc[:s_per, :] + racc[:, :].astype(jnp.float32)
        out_ref[:, :] = aug0[:, :d] / aug0[:, d:d + 1]

        for desc in [q_desc, c_desc] + r_desc + l_desc:
            desc.wait_send()

    return pl.pallas_call(
        body,
        out_shape=jax.ShapeDtypeStruct((s_per, d), jnp.float32),
        in_specs=[
            pl.BlockSpec(memory_space=pltpu.VMEM),
            pl.BlockSpec(memory_space=pltpu.VMEM),
            pl.BlockSpec(memory_space=pltpu.VMEM),
        ],
        out_specs=pl.BlockSpec(memory_space=pltpu.VMEM),
        scratch_shapes=[
            pltpu.VMEM((s_per, d), jnp.bfloat16),
            pltpu.VMEM((s_per, d), jnp.bfloat16),
            pltpu.VMEM((R_HOPS + 1, 2 * s_per, d), jnp.bfloat16),
            pltpu.VMEM((L_HOPS + 1, 2 * s_per, d), jnp.bfloat16),
            pltpu.VMEM((s_per, 2 * d), jnp.bfloat16),
            pltpu.VMEM((s_per, 2 * d), jnp.bfloat16),
            pltpu.SemaphoreType.DMA((2,)),
            pltpu.SemaphoreType.DMA((R_HOPS + 1,)),
            pltpu.SemaphoreType.DMA((R_HOPS + 1,)),
            pltpu.SemaphoreType.DMA((L_HOPS + 1,)),
            pltpu.SemaphoreType.DMA((L_HOPS + 1,)),
            pltpu.SemaphoreType.DMA((2,)),
        ],
        compiler_params=pltpu.CompilerParams(collective_id=0),
    )(q, k, v)


# device time: 53665 ns/iter; 1.1998x vs baseline; 1.1634x over previous
import jax
import jax.numpy as jnp
from jax import lax
from jax.experimental import pallas as pl
from jax.experimental.pallas import tpu as pltpu

N_DEV = 32
ROW = 8
COL = 4
R_HOPS = ROW // 2
L_HOPS = ROW // 2 - 1


def kernel(q, k, v):
    s_per, d = q.shape
    scale = 1.0 / (d ** 0.5)

    def body(q_ref, k_ref, v_ref, out_ref,
             qa, rbuf, lbuf, sacc1, racc1, sacc2, racc2,
             qa_sems, br_send, br_recv, bl_send, bl_recv, c1_sems, c2_sems):
        p = lax.axis_index("i")
        base = (p // ROW) * ROW
        w = p % ROW
        z = p // ROW
        right = base + (w + 1) % ROW
        left = base + (w - 1) % ROW
        pz1 = (z ^ 1) * ROW + w
        pz2 = (z ^ 2) * ROW + w

        barrier_sem = pltpu.get_barrier_semaphore()
        for nbr in [left, right, pz1, pz2]:
            pl.semaphore_signal(
                barrier_sem, inc=1,
                device_id=(nbr,), device_id_type=pl.DeviceIdType.MESH,
            )
        pl.semaphore_wait(barrier_sem, 4)

        k_bf = k_ref[:, :].astype(jnp.bfloat16)
        v_bf = v_ref[:, :].astype(jnp.bfloat16)
        rbuf[0, :s_per, :] = k_bf
        rbuf[0, s_per:, :] = v_bf
        lbuf[0, :s_per, :] = k_bf
        lbuf[0, s_per:, :] = v_bf
        qa[0, :, :] = (q_ref[:, :] * scale).astype(jnp.bfloat16)

        def make(src, dst, send, recv, dev):
            return pltpu.make_async_remote_copy(
                src_ref=src, dst_ref=dst, send_sem=send, recv_sem=recv,
                device_id=(dev,), device_id_type=pl.DeviceIdType.MESH,
            )

        q1_desc = make(qa.at[0], qa.at[1], qa_sems.at[0], qa_sems.at[1], pz1)
        q2a_desc = make(qa.at[0], qa.at[2], qa_sems.at[2], qa_sems.at[3], pz2)
        q2b_desc = make(qa.at[1], qa.at[3], qa_sems.at[4], qa_sems.at[5], pz2)
        r_desc = [
            make(rbuf.at[h], rbuf.at[h + 1], br_send.at[h + 1],
                 br_recv.at[h + 1], right)
            for h in range(R_HOPS)
        ]
        l_desc = [
            make(lbuf.at[h], lbuf.at[h + 1], bl_send.at[h + 1],
                 bl_recv.at[h + 1], left)
            for h in range(L_HOPS)
        ]
        c1_desc = make(sacc1, racc1, c1_sems.at[0], c1_sems.at[1], pz2)
        c2_desc = make(sacc2, racc2, c2_sems.at[0], c2_sems.at[1], pz1)

        r_desc[0].start()
        l_desc[0].start()
        q1_desc.start()
        q1_desc.wait_recv()
        q2a_desc.start()
        q2b_desc.start()
        q2a_desc.wait_recv()
        q2b_desc.wait_recv()

        q_all = jnp.concatenate(
            [qa[0, :, :], qa[1, :, :], qa[2, :, :], qa[3, :, :]], axis=0
        )
        ones = jnp.ones((s_per, d), dtype=jnp.bfloat16)
        acc = jnp.zeros((COL * s_per, 2 * d), dtype=jnp.float32)

        def fold(acc, buf, slot):
            k_h = buf[slot, :s_per, :]
            v_aug = jnp.concatenate([buf[slot, s_per:, :], ones], axis=1)
            scores = jax.lax.dot_general(
                q_all, k_h,
                (((1,), (1,)), ((), ())),
                preferred_element_type=jnp.float32,
            )
            pr = jnp.exp(scores).astype(jnp.bfloat16)
            pv = jax.lax.dot_general(
                pr, v_aug,
                (((1,), (0,)), ((), ())),
                preferred_element_type=jnp.float32,
            )
            return acc + pv

        acc = fold(acc, rbuf, 0)

        for h in range(1, R_HOPS + 1):
            r_desc[h - 1].wait_recv()
            if h < R_HOPS:
                r_desc[h].start()
            acc = fold(acc, rbuf, h)
            if h <= L_HOPS:
                l_desc[h - 1].wait_recv()
                if h < L_HOPS:
                    l_desc[h].start()
                acc = fold(acc, lbuf, h)

        sacc1[:, :] = acc[2 * s_per:, :].astype(jnp.bfloat16)
        c1_desc.start()
        c1_desc.wait_recv()
        acc01 = acc[:2 * s_per, :] + racc1[:, :].astype(jnp.float32)
        sacc2[:, :] = acc01[s_per:, :].astype(jnp.bfloat16)
        c2_desc.start()
        c2_desc.wait_recv()
        aug0 = acc01[:s_per, :] + racc2[:, :].astype(jnp.float32)
        out_ref[:, :] = aug0[:, :d] / aug0[:, d:d + 1]

        for desc in [q1_desc, q2a_desc, q2b_desc, c1_desc, c2_desc] \
                + r_desc + l_desc:
            desc.wait_send()

    return pl.pallas_call(
        body,
        out_shape=jax.ShapeDtypeStruct((s_per, d), jnp.float32),
        in_specs=[
            pl.BlockSpec(memory_space=pltpu.VMEM),
            pl.BlockSpec(memory_space=pltpu.VMEM),
            pl.BlockSpec(memory_space=pltpu.VMEM),
        ],
        out_specs=pl.BlockSpec(memory_space=pltpu.VMEM),
        scratch_shapes=[
            pltpu.VMEM((COL, s_per, d), jnp.bfloat16),
            pltpu.VMEM((R_HOPS + 1, 2 * s_per, d), jnp.bfloat16),
            pltpu.VMEM((L_HOPS + 1, 2 * s_per, d), jnp.bfloat16),
            pltpu.VMEM((2 * s_per, 2 * d), jnp.bfloat16),
            pltpu.VMEM((2 * s_per, 2 * d), jnp.bfloat16),
            pltpu.VMEM((s_per, 2 * d), jnp.bfloat16),
            pltpu.VMEM((s_per, 2 * d), jnp.bfloat16),
            pltpu.SemaphoreType.DMA((6,)),
            pltpu.SemaphoreType.DMA((R_HOPS + 1,)),
            pltpu.SemaphoreType.DMA((R_HOPS + 1,)),
            pltpu.SemaphoreType.DMA((L_HOPS + 1,)),
            pltpu.SemaphoreType.DMA((L_HOPS + 1,)),
            pltpu.SemaphoreType.DMA((2,)),
            pltpu.SemaphoreType.DMA((2,)),
        ],
        compiler_params=pltpu.CompilerParams(collective_id=0),
    )(q, k, v)


# device time: 46411 ns/iter; 1.3873x vs baseline; 1.1563x over previous
import jax
import jax.numpy as jnp
from jax import lax
from jax.experimental import pallas as pl
from jax.experimental.pallas import tpu as pltpu

N_DEV = 32
ROW = 8
COL = 4
R_HOPS = ROW // 2
L_HOPS = ROW // 2 - 1


def kernel(q, k, v):
    s_per, d = q.shape
    scale = 1.0 / (d ** 0.5)

    def body(q_ref, k_ref, v_ref, out_ref,
             qa, rbuf, lbuf, sacc1, racc1, sacc2, racc2,
             qa_sems, br_send, br_recv, bl_send, bl_recv, c1_sems, c2_sems):
        p = lax.axis_index("i")
        base = (p // ROW) * ROW
        w = p % ROW
        z = p // ROW
        right = base + (w + 1) % ROW
        left = base + (w - 1) % ROW
        pz1 = (z ^ 1) * ROW + w
        pz2 = (z ^ 2) * ROW + w

        barrier_sem = pltpu.get_barrier_semaphore()
        for nbr in [left, right, pz1, pz2]:
            pl.semaphore_signal(
                barrier_sem, inc=1,
                device_id=(nbr,), device_id_type=pl.DeviceIdType.MESH,
            )
        pl.semaphore_wait(barrier_sem, 4)

        k_bf = k_ref[:, :].astype(jnp.bfloat16)
        v_bf = v_ref[:, :].astype(jnp.bfloat16)
        rbuf[0, :s_per, :] = k_bf
        rbuf[0, s_per:, :] = v_bf
        lbuf[0, :s_per, :] = k_bf
        lbuf[0, s_per:, :] = v_bf
        qa[0, :, :] = (q_ref[:, :] * scale).astype(jnp.bfloat16)

        def make(src, dst, send, recv, dev):
            return pltpu.make_async_remote_copy(
                src_ref=src, dst_ref=dst, send_sem=send, recv_sem=recv,
                device_id=(dev,), device_id_type=pl.DeviceIdType.MESH,
            )

        q1_desc = make(qa.at[0], qa.at[1], qa_sems.at[0], qa_sems.at[1], pz1)
        q2a_desc = make(qa.at[0], qa.at[2], qa_sems.at[2], qa_sems.at[3], pz2)
        q2b_desc = make(qa.at[1], qa.at[3], qa_sems.at[4], qa_sems.at[5], pz2)
        r_desc = [
            make(rbuf.at[h], rbuf.at[h + 1], br_send.at[h + 1],
                 br_recv.at[h + 1], right)
            for h in range(R_HOPS)
        ]
        l_desc = [
            make(lbuf.at[h], lbuf.at[h + 1], bl_send.at[h + 1],
                 bl_recv.at[h + 1], left)
            for h in range(L_HOPS)
        ]
        c1_desc = make(sacc1, racc1, c1_sems.at[0], c1_sems.at[1], pz2)
        c2_desc = make(sacc2, racc2, c2_sems.at[0], c2_sems.at[1], pz1)

        r_desc[0].start()
        l_desc[0].start()
        q1_desc.start()

        ones = jnp.ones((s_per, d), dtype=jnp.bfloat16)

        def fold(q_blk, buf, slot):
            k_h = buf[slot, :s_per, :]
            v_aug = jnp.concatenate([buf[slot, s_per:, :], ones], axis=1)
            scores = jax.lax.dot_general(
                q_blk, k_h,
                (((1,), (1,)), ((), ())),
                preferred_element_type=jnp.float32,
            )
            pr = jnp.exp(scores).astype(jnp.bfloat16)
            return jax.lax.dot_general(
                pr, v_aug,
                (((1,), (0,)), ((), ())),
                preferred_element_type=jnp.float32,
            )

        q1_desc.wait_recv()
        q2a_desc.start()
        q2b_desc.start()
        q01 = jnp.concatenate([qa[0, :, :], qa[1, :, :]], axis=0)
        acc01 = fold(q01, rbuf, 0)

        r_desc[0].wait_recv()
        r_desc[1].start()
        acc01 = acc01 + fold(q01, rbuf, 1)
        l_desc[0].wait_recv()
        l_desc[1].start()
        acc01 = acc01 + fold(q01, lbuf, 1)

        q2a_desc.wait_recv()
        q2b_desc.wait_recv()
        q23 = jnp.concatenate([qa[2, :, :], qa[3, :, :]], axis=0)
        acc23 = fold(q23, rbuf, 0) + fold(q23, rbuf, 1) + fold(q23, lbuf, 1)

        for h in range(2, R_HOPS):
            r_desc[h - 1].wait_recv()
            r_desc[h].start()
            acc01 = acc01 + fold(q01, rbuf, h)
            acc23 = acc23 + fold(q23, rbuf, h)
            l_desc[h - 1].wait_recv()
            if h < L_HOPS:
                l_desc[h].start()
            acc01 = acc01 + fold(q01, lbuf, h)
            acc23 = acc23 + fold(q23, lbuf, h)

        r_desc[R_HOPS - 1].wait_recv()
        acc23 = acc23 + fold(q23, rbuf, R_HOPS)
        sacc1[:, :] = acc23.astype(jnp.bfloat16)
        c1_desc.start()
        acc01 = acc01 + fold(q01, rbuf, R_HOPS)
        c1_desc.wait_recv()
        acc01 = acc01 + racc1[:, :].astype(jnp.float32)
        sacc2[:, :] = acc01[s_per:, :].astype(jnp.bfloat16)
        c2_desc.start()
        c2_desc.wait_recv()
        aug0 = acc01[:s_per, :] + racc2[:, :].astype(jnp.float32)
        out_ref[:, :] = aug0[:, :d] / aug0[:, d:d + 1]

        for desc in [q1_desc, q2a_desc, q2b_desc, c1_desc, c2_desc] \
                + r_desc + l_desc:
            desc.wait_send()

    return pl.pallas_call(
        body,
        out_shape=jax.ShapeDtypeStruct((s_per, d), jnp.float32),
        in_specs=[
            pl.BlockSpec(memory_space=pltpu.VMEM),
            pl.BlockSpec(memory_space=pltpu.VMEM),
            pl.BlockSpec(memory_space=pltpu.VMEM),
        ],
        out_specs=pl.BlockSpec(memory_space=pltpu.VMEM),
        scratch_shapes=[
            pltpu.VMEM((COL, s_per, d), jnp.bfloat16),
            pltpu.VMEM((R_HOPS + 1, 2 * s_per, d), jnp.bfloat16),
            pltpu.VMEM((L_HOPS + 1, 2 * s_per, d), jnp.bfloat16),
            pltpu.VMEM((2 * s_per, 2 * d), jnp.bfloat16),
            pltpu.VMEM((2 * s_per, 2 * d), jnp.bfloat16),
            pltpu.VMEM((s_per, 2 * d), jnp.bfloat16),
            pltpu.VMEM((s_per, 2 * d), jnp.bfloat16),
            pltpu.SemaphoreType.DMA((6,)),
            pltpu.SemaphoreType.DMA((R_HOPS + 1,)),
            pltpu.SemaphoreType.DMA((R_HOPS + 1,)),
            pltpu.SemaphoreType.DMA((L_HOPS + 1,)),
            pltpu.SemaphoreType.DMA((L_HOPS + 1,)),
            pltpu.SemaphoreType.DMA((2,)),
            pltpu.SemaphoreType.DMA((2,)),
        ],
        compiler_params=pltpu.CompilerParams(collective_id=0),
    )(q, k, v)


# device time: 43529 ns/iter; 1.4792x vs baseline; 1.0662x over previous
import jax
import jax.numpy as jnp
from jax import lax
from jax.experimental import pallas as pl
from jax.experimental.pallas import tpu as pltpu

N_DEV = 32
ROW = 8
COL = 4
R_HOPS = ROW // 2
L_HOPS = ROW // 2 - 1


def kernel(q, k, v):
    s_per, d = q.shape
    scale = 1.0 / (d ** 0.5)

    def body(q_ref, k_ref, v_ref, out_ref,
             qa, rbuf, lbuf, sblk, rblk,
             qa_sems, br_send, br_recv, bl_send, bl_recv, cs_sems, cr_sems):
        p = lax.axis_index("i")
        base = (p // ROW) * ROW
        w = p % ROW
        z = p // ROW
        right = base + (w + 1) % ROW
        left = base + (w - 1) % ROW
        pz1 = (z ^ 1) * ROW + w
        pz2 = (z ^ 2) * ROW + w
        pz3 = (z ^ 3) * ROW + w

        barrier_sem = pltpu.get_barrier_semaphore()
        for nbr in [left, right, pz1, pz2, pz3]:
            pl.semaphore_signal(
                barrier_sem, inc=1,
                device_id=(nbr,), device_id_type=pl.DeviceIdType.MESH,
            )
        pl.semaphore_wait(barrier_sem, 5)

        k_bf = k_ref[:, :].astype(jnp.bfloat16)
        v_bf = v_ref[:, :].astype(jnp.bfloat16)
        rbuf[0, :s_per, :] = k_bf
        rbuf[0, s_per:, :] = v_bf
        lbuf[0, :s_per, :] = k_bf
        lbuf[0, s_per:, :] = v_bf
        qa[0, :, :] = (q_ref[:, :] * scale).astype(jnp.bfloat16)

        def make(src, dst, send, recv, dev):
            return pltpu.make_async_remote_copy(
                src_ref=src, dst_ref=dst, send_sem=send, recv_sem=recv,
                device_id=(dev,), device_id_type=pl.DeviceIdType.MESH,
            )

        q1_desc = make(qa.at[0], qa.at[1], qa_sems.at[0], qa_sems.at[1], pz1)
        q2a_desc = make(qa.at[0], qa.at[2], qa_sems.at[2], qa_sems.at[3], pz2)
        q2b_desc = make(qa.at[1], qa.at[3], qa_sems.at[4], qa_sems.at[5], pz2)
        r_desc = [
            make(rbuf.at[h], rbuf.at[h + 1], br_send.at[h + 1],
                 br_recv.at[h + 1], right)
            for h in range(R_HOPS)
        ]
        l_desc = [
            make(lbuf.at[h], lbuf.at[h + 1], bl_send.at[h + 1],
                 bl_recv.at[h + 1], left)
            for h in range(L_HOPS)
        ]
        c_desc = [
            make(sblk.at[s - 1], rblk.at[s - 1],
                 cs_sems.at[s - 1], cr_sems.at[s - 1],
                 (z ^ s) * ROW + w)
            for s in (1, 2, 3)
        ]

        r_desc[0].start()
        l_desc[0].start()
        q1_desc.start()

        ones = jnp.ones((s_per, d), dtype=jnp.bfloat16)

        def fold(q_blk, buf, slot):
            k_h = buf[slot, :s_per, :]
            v_aug = jnp.concatenate([buf[slot, s_per:, :], ones], axis=1)
            scores = jax.lax.dot_general(
                q_blk, k_h,
                (((1,), (1,)), ((), ())),
                preferred_element_type=jnp.float32,
            )
            pr = jnp.exp(scores).astype(jnp.bfloat16)
            return jax.lax.dot_general(
                pr, v_aug,
                (((1,), (0,)), ((), ())),
                preferred_element_type=jnp.float32,
            )

        q1_desc.wait_recv()
        q2a_desc.start()
        q2b_desc.start()
        q01 = jnp.concatenate([qa[0, :, :], qa[1, :, :]], axis=0)
        acc01 = fold(q01, rbuf, 0)

        r_desc[0].wait_recv()
        r_desc[1].start()
        acc01 = acc01 + fold(q01, rbuf, 1)
        l_desc[0].wait_recv()
        l_desc[1].start()
        acc01 = acc01 + fold(q01, lbuf, 1)

        q2a_desc.wait_recv()
        q2b_desc.wait_recv()
        q23 = jnp.concatenate([qa[2, :, :], qa[3, :, :]], axis=0)
        acc23 = fold(q23, rbuf, 0) + fold(q23, rbuf, 1) + fold(q23, lbuf, 1)

        for h in range(2, R_HOPS):
            r_desc[h - 1].wait_recv()
            r_desc[h].start()
            acc01 = acc01 + fold(q01, rbuf, h)
            acc23 = acc23 + fold(q23, rbuf, h)
            l_desc[h - 1].wait_recv()
            if h < L_HOPS:
                l_desc[h].start()
            acc01 = acc01 + fold(q01, lbuf, h)
            acc23 = acc23 + fold(q23, lbuf, h)

        r_desc[R_HOPS - 1].wait_recv()
        acc23 = acc23 + fold(q23, rbuf, R_HOPS)
        sblk[1, :, :] = acc23[:s_per, :].astype(jnp.bfloat16)
        sblk[2, :, :] = acc23[s_per:, :].astype(jnp.bfloat16)
        c_desc[1].start()
        c_desc[2].start()
        acc01 = acc01 + fold(q01, rbuf, R_HOPS)
        sblk[0, :, :] = acc01[s_per:, :].astype(jnp.bfloat16)
        c_desc[0].start()
        c_desc[0].wait_recv()
        c_desc[1].wait_recv()
        c_desc[2].wait_recv()
        aug0 = (acc01[:s_per, :]
                + rblk[0, :, :].astype(jnp.float32)
                + rblk[1, :, :].astype(jnp.float32)
                + rblk[2, :, :].astype(jnp.float32))
        out_ref[:, :] = aug0[:, :d] / aug0[:, d:d + 1]

        for desc in [q1_desc, q2a_desc, q2b_desc] + c_desc \
                + r_desc + l_desc:
            desc.wait_send()

    return pl.pallas_call(
        body,
        out_shape=jax.ShapeDtypeStruct((s_per, d), jnp.float32),
        in_specs=[
            pl.BlockSpec(memory_space=pltpu.VMEM),
            pl.BlockSpec(memory_space=pltpu.VMEM),
            pl.BlockSpec(memory_space=pltpu.VMEM),
        ],
        out_specs=pl.BlockSpec(memory_space=pltpu.VMEM),
        scratch_shapes=[
            pltpu.VMEM((COL, s_per, d), jnp.bfloat16),
            pltpu.VMEM((R_HOPS + 1, 2 * s_per, d), jnp.bfloat16),
            pltpu.VMEM((L_HOPS + 1, 2 * s_per, d), jnp.bfloat16),
            pltpu.VMEM((3, s_per, 2 * d), jnp.bfloat16),
            pltpu.VMEM((3, s_per, 2 * d), jnp.bfloat16),
            pltpu.SemaphoreType.DMA((6,)),
            pltpu.SemaphoreType.DMA((R_HOPS + 1,)),
            pltpu.SemaphoreType.DMA((R_HOPS + 1,)),
            pltpu.SemaphoreType.DMA((L_HOPS + 1,)),
            pltpu.SemaphoreType.DMA((L_HOPS + 1,)),
            pltpu.SemaphoreType.DMA((3,)),
            pltpu.SemaphoreType.DMA((3,)),
        ],
        compiler_params=pltpu.CompilerParams(collective_id=0),
    )(q, k, v)


# device time: 38177 ns/iter; 1.6865x vs baseline; 1.1402x over previous
import jax
import jax.numpy as jnp
from jax import lax
from jax.experimental import pallas as pl
from jax.experimental.pallas import tpu as pltpu

N_DEV = 32
ROW = 8
COL = 4
R_HOPS = ROW // 2
L_HOPS = ROW // 2 - 1


def kernel(q, k, v):
    s_per, d = q.shape
    scale = 1.0 / (d ** 0.5)

    def body(q_ref, k_ref, v_ref, out_ref,
             qa, rbuf, lbuf, sblk, rblk, sblk_l, rblk_l,
             qa_sems, br_send, br_recv, bl_send, bl_recv,
             cs_sems, cr_sems, cls_sems, clr_sems):
        p = lax.axis_index("i")
        base = (p // ROW) * ROW
        w = p % ROW
        z = p // ROW
        right = base + (w + 1) % ROW
        left = base + (w - 1) % ROW
        pz1 = (z ^ 1) * ROW + w
        pz2 = (z ^ 2) * ROW + w
        pz3 = (z ^ 3) * ROW + w

        barrier_sem = pltpu.get_barrier_semaphore()
        for nbr in [left, right, pz1, pz2, pz3]:
            pl.semaphore_signal(
                barrier_sem, inc=1,
                device_id=(nbr,), device_id_type=pl.DeviceIdType.MESH,
            )
        pl.semaphore_wait(barrier_sem, 5)

        k_bf = k_ref[:, :].astype(jnp.bfloat16)
        v_bf = v_ref[:, :].astype(jnp.bfloat16)
        rbuf[0, :s_per, :] = k_bf
        rbuf[0, s_per:, :] = v_bf
        lbuf[0, :s_per, :] = k_bf
        lbuf[0, s_per:, :] = v_bf
        qa[0, :, :] = (q_ref[:, :] * scale).astype(jnp.bfloat16)

        def make(src, dst, send, recv, dev):
            return pltpu.make_async_remote_copy(
                src_ref=src, dst_ref=dst, send_sem=send, recv_sem=recv,
                device_id=(dev,), device_id_type=pl.DeviceIdType.MESH,
            )

        q1_desc = make(qa.at[0], qa.at[1], qa_sems.at[0], qa_sems.at[1], pz1)
        q2a_desc = make(qa.at[0], qa.at[2], qa_sems.at[2], qa_sems.at[3], pz2)
        q2b_desc = make(qa.at[1], qa.at[3], qa_sems.at[4], qa_sems.at[5], pz2)
        r_desc = [
            make(rbuf.at[h], rbuf.at[h + 1], br_send.at[h + 1],
                 br_recv.at[h + 1], right)
            for h in range(R_HOPS)
        ]
        l_desc = [
            make(lbuf.at[h], lbuf.at[h + 1], bl_send.at[h + 1],
                 bl_recv.at[h + 1], left)
            for h in range(L_HOPS)
        ]
        c_desc = [
            make(sblk.at[s - 1], rblk.at[s - 1],
                 cs_sems.at[s - 1], cr_sems.at[s - 1],
                 (z ^ s) * ROW + w)
            for s in (1, 2, 3)
        ]
        cl_desc = [
            make(sblk_l.at[s - 1], rblk_l.at[s - 1],
                 cls_sems.at[s - 1], clr_sems.at[s - 1],
                 (z ^ s) * ROW + w)
            for s in (1, 2, 3)
        ]

        r_desc[0].start()
        l_desc[0].start()
        q1_desc.start()

        ones = jnp.ones((s_per, d), dtype=jnp.bfloat16)

        def fold(q_blk, buf, slot):
            k_h = buf[slot, :s_per, :]
            v_aug = jnp.concatenate([buf[slot, s_per:, :], ones], axis=1)
            scores = jax.lax.dot_general(
                q_blk, k_h,
                (((1,), (1,)), ((), ())),
                preferred_element_type=jnp.float32,
            )
            pr = jnp.exp(scores).astype(jnp.bfloat16)
            return jax.lax.dot_general(
                pr, v_aug,
                (((1,), (0,)), ((), ())),
                preferred_element_type=jnp.float32,
            )

        q1_desc.wait_recv()
        q2a_desc.start()
        q2b_desc.start()
        q01 = jnp.concatenate([qa[0, :, :], qa[1, :, :]], axis=0)
        acc01 = fold(q01, rbuf, 0)

        r_desc[0].wait_recv()
        r_desc[1].start()
        acc01 = acc01 + fold(q01, rbuf, 1)
        l_desc[0].wait_recv()
        l_desc[1].start()
        acc01 = acc01 + fold(q01, lbuf, 1)

        q2a_desc.wait_recv()
        q2b_desc.wait_recv()
        q23 = jnp.concatenate([qa[2, :, :], qa[3, :, :]], axis=0)
        acc23 = fold(q23, rbuf, 0) + fold(q23, rbuf, 1) + fold(q23, lbuf, 1)

        for h in range(2, R_HOPS):
            r_desc[h - 1].wait_recv()
            r_desc[h].start()
            acc01 = acc01 + fold(q01, rbuf, h)
            acc23 = acc23 + fold(q23, rbuf, h)
            l_desc[h - 1].wait_recv()
            if h < L_HOPS:
                l_desc[h].start()
            acc01 = acc01 + fold(q01, lbuf, h)
            acc23 = acc23 + fold(q23, lbuf, h)

        def pack(blk):
            return (
                blk[:, :d].astype(jnp.bfloat16),
                jnp.reshape(blk[:, d], (s_per // d, d)),
            )

        r_desc[R_HOPS - 1].wait_recv()
        acc23 = acc23 + fold(q23, rbuf, R_HOPS)
        sblk[1, :, :], sblk_l[1, :, :] = pack(acc23[:s_per, :])
        sblk[2, :, :], sblk_l[2, :, :] = pack(acc23[s_per:, :])
        c_desc[1].start()
        c_desc[2].start()
        cl_desc[1].start()
        cl_desc[2].start()
        acc01 = acc01 + fold(q01, rbuf, R_HOPS)
        sblk[0, :, :], sblk_l[0, :, :] = pack(acc01[s_per:, :])
        c_desc[0].start()
        cl_desc[0].start()
        for desc in c_desc + cl_desc:
            desc.wait_recv()
        aug_v = (acc01[:s_per, :d]
                 + rblk[0, :, :].astype(jnp.float32)
                 + rblk[1, :, :].astype(jnp.float32)
                 + rblk[2, :, :].astype(jnp.float32))
        l_pack = (jnp.reshape(acc01[:s_per, d], (s_per // d, d))
                  + rblk_l[0, :, :] + rblk_l[1, :, :] + rblk_l[2, :, :])
        out_ref[:, :] = aug_v / jnp.reshape(l_pack, (s_per, 1))

        for desc in [q1_desc, q2a_desc, q2b_desc] + c_desc + cl_desc \
                + r_desc + l_desc:
            desc.wait_send()

    return pl.pallas_call(
        body,
        out_shape=jax.ShapeDtypeStruct((s_per, d), jnp.float32),
        in_specs=[
            pl.BlockSpec(memory_space=pltpu.VMEM),
            pl.BlockSpec(memory_space=pltpu.VMEM),
            pl.BlockSpec(memory_space=pltpu.VMEM),
        ],
        out_specs=pl.BlockSpec(memory_space=pltpu.VMEM),
        scratch_shapes=[
            pltpu.VMEM((COL, s_per, d), jnp.bfloat16),
            pltpu.VMEM((R_HOPS + 1, 2 * s_per, d), jnp.bfloat16),
            pltpu.VMEM((L_HOPS + 1, 2 * s_per, d), jnp.bfloat16),
            pltpu.VMEM((3, s_per, d), jnp.bfloat16),
            pltpu.VMEM((3, s_per, d), jnp.bfloat16),
            pltpu.VMEM((3, s_per // d, d), jnp.float32),
            pltpu.VMEM((3, s_per // d, d), jnp.float32),
            pltpu.SemaphoreType.DMA((6,)),
            pltpu.SemaphoreType.DMA((R_HOPS + 1,)),
            pltpu.SemaphoreType.DMA((R_HOPS + 1,)),
            pltpu.SemaphoreType.DMA((L_HOPS + 1,)),
            pltpu.SemaphoreType.DMA((L_HOPS + 1,)),
            pltpu.SemaphoreType.DMA((3,)),
            pltpu.SemaphoreType.DMA((3,)),
            pltpu.SemaphoreType.DMA((3,)),
            pltpu.SemaphoreType.DMA((3,)),
        ],
        compiler_params=pltpu.CompilerParams(collective_id=0),
    )(q, k, v)
